# baseline (speedup 1.0000x reference)
import functools
import jax
import jax.numpy as jnp
from jax import lax
from jax.experimental import pallas as pl
from jax.experimental.pallas import tpu as pltpu
from jax.experimental.pallas import tpu_sc as plsc

D_MODEL = 64
DPAD = 128
N_VOC = 1000000
N_SUPER = 977 * 512
SCALE = 8.0
NC = 2
NS = 16
NW = NC * NS
LANES = 16

B_TOTAL = 4096 * 200
BPW = B_TOTAL // NW
CHUNK = 400
NCHUNK = BPW // CHUNK
NBUF = 2

REPACK_BLK = 1024


def _repack_body(in_ref, o_ref):
    a = in_ref[...]
    rows_i = lax.broadcasted_iota(jnp.int32, (D_MODEL, D_MODEL), 0)
    cols_i = lax.broadcasted_iota(jnp.int32, (D_MODEL, D_MODEL), 1)
    eye = jnp.where(rows_i == cols_i, 1.0, 0.0).astype(jnp.float32)
    at = lax.dot_general(a, eye, (((0,), (0,)), ((), ())),
                         precision=lax.Precision.HIGHEST,
                         preferred_element_type=jnp.float32)
    half = REPACK_BLK // 2
    o_ref[...] = jnp.concatenate([at[:half], at[half:]], axis=1)


def _emb_body(x_hbm, lut_hbm, out_hbm, idx_v, idx2a, idx2b, rows0, rows1,
              gsem0, gsem1, ssem0, ssem1):
    cid = lax.axis_index("c")
    sid = lax.axis_index("s")
    wid = sid * NC + cid
    base = wid * BPW

    pltpu.sync_copy(x_hbm.at[pl.ds(base, BPW)], idx_v)

    rows = (rows0, rows1)
    idx2 = (idx2a, idx2b)
    gsem = (gsem0, gsem1)
    ssem = (ssem0, ssem1)

    def start_gather(b, off):
        def shift(i, c):
            sl = pl.ds(i * LANES, LANES)
            v = idx_v[pl.ds(off + i * LANES, LANES)]
            idx2[b][sl] = (
                lax.shift_left(lax.shift_right_logical(v, 10), 9) | (v & 511))
            return c

        lax.fori_loop(0, CHUNK // LANES, shift, 0, unroll=4)
        pltpu.async_copy(lut_hbm.at[idx2[b]], rows[b], gsem[b])

    def wait_gather(b):
        pltpu.make_async_copy(lut_hbm.at[idx2[b]], rows[b], gsem[b]).wait()

    def start_scatter(b, off):
        pltpu.async_copy(rows[b], out_hbm.at[pl.ds(base + off, CHUNK)],
                         ssem[b])

    def wait_scatter(b, off):
        pltpu.make_async_copy(rows[b], out_hbm.at[pl.ds(base + off, CHUNK)],
                              ssem[b]).wait()

    for b in range(NBUF):
        start_gather(b, b * CHUNK)

    def group(g, carry):
        for b in range(NBUF):
            off = (g * NBUF + b) * CHUNK
            wait_gather(b)

            def select_group(g2, c):
                hv = lax.shift_right_logical(
                    idx_v[pl.ds(off + g2 * LANES, LANES)], 9) & 1
                for k in range(LANES):
                    r = g2 * LANES + k
                    hk = hv[k] > 0
                    for j in range(D_MODEL // LANES):
                        lo = rows[b][r, pl.ds(j * LANES, LANES)]
                        hi = rows[b][r, pl.ds(D_MODEL + j * LANES, LANES)]
                        rows[b][r, pl.ds(j * LANES, LANES)] = (
                            jnp.where(hk, hi, lo) * SCALE)
                return c

            lax.fori_loop(0, CHUNK // LANES, select_group, 0)
            start_scatter(b, off)

            nxt = off + NBUF * CHUNK

            @pl.when(g * NBUF + b + NBUF < NCHUNK)
            def _():
                wait_scatter(b, off)
                start_gather(b, nxt)
        return carry

    lax.fori_loop(0, NCHUNK // NBUF, group, 0)

    for b in range(NBUF):
        off = (NCHUNK - NBUF + b) * CHUNK
        wait_scatter(b, off)


@jax.jit
def _run(x_flat, lut_t):
    lut2 = pl.pallas_call(
        _repack_body,
        grid=(pl.cdiv(N_VOC, REPACK_BLK),),
        in_specs=[pl.BlockSpec((D_MODEL, REPACK_BLK), lambda i: (0, i))],
        out_specs=pl.BlockSpec((REPACK_BLK // 2, DPAD), lambda i: (i, 0)),
        out_shape=jax.ShapeDtypeStruct((N_SUPER, DPAD), jnp.float32),
    )(lut_t)

    mesh = plsc.VectorSubcoreMesh(core_axis_name="c", subcore_axis_name="s")
    k = functools.partial(
        pl.kernel,
        mesh=mesh,
        out_type=jax.ShapeDtypeStruct((B_TOTAL, DPAD), jnp.float32),
        compiler_params=pltpu.CompilerParams(use_tc_tiling_on_sc=True),
        scratch_types=[
            pltpu.VMEM((BPW,), jnp.int32),
            pltpu.VMEM((CHUNK,), jnp.int32),
            pltpu.VMEM((CHUNK,), jnp.int32),
            pltpu.VMEM((CHUNK, DPAD), jnp.float32),
            pltpu.VMEM((CHUNK, DPAD), jnp.float32),
            pltpu.SemaphoreType.DMA,
            pltpu.SemaphoreType.DMA,
            pltpu.SemaphoreType.DMA,
            pltpu.SemaphoreType.DMA,
        ],
    )(_emb_body)
    return k(x_flat, lut2)


def kernel(x, lut):
    xf = x.reshape(-1).astype(jnp.int32)
    out = _run(xf, lut.T)
    return out[:, :D_MODEL].reshape(x.shape[0], x.shape[1], D_MODEL)


# speedup vs baseline: 1.6717x; 1.6717x over previous
import functools
import jax
import jax.numpy as jnp
from jax import lax
from jax.experimental import pallas as pl
from jax.experimental.pallas import tpu as pltpu
from jax.experimental.pallas import tpu_sc as plsc

D_MODEL = 64
DPAD = 128
SCALE = 8.0
NC = 2
NS = 16
NW = NC * NS
LANES = 16

B_TOTAL = 4096 * 200
BPW = B_TOTAL // NW
CHUNK = 200
NCHUNK = BPW // CHUNK
NBUF = 4


def _emb_body(x_hbm, lut_hbm, out_hbm, idx_v, rows0, rows1, rows2, rows3,
              gsem0, gsem1, gsem2, gsem3, ssem0, ssem1, ssem2, ssem3):
    cid = lax.axis_index("c")
    sid = lax.axis_index("s")
    wid = sid * NC + cid
    base = wid * BPW

    pltpu.sync_copy(x_hbm.at[pl.ds(base, BPW)], idx_v)

    rows = (rows0, rows1, rows2, rows3)
    gsem = (gsem0, gsem1, gsem2, gsem3)
    ssem = (ssem0, ssem1, ssem2, ssem3)

    def start_gather(b, off):
        pltpu.async_copy(lut_hbm.at[idx_v.at[pl.ds(off, CHUNK)]], rows[b],
                         gsem[b])

    def wait_gather(b, off):
        pltpu.make_async_copy(lut_hbm.at[idx_v.at[pl.ds(off, CHUNK)]],
                              rows[b], gsem[b]).wait()

    def start_scatter(b, off):
        pltpu.async_copy(rows[b], out_hbm.at[pl.ds(base + off, CHUNK)],
                         ssem[b])

    def wait_scatter(b, off):
        pltpu.make_async_copy(rows[b], out_hbm.at[pl.ds(base + off, CHUNK)],
                              ssem[b]).wait()

    for b in range(NBUF):
        start_gather(b, b * CHUNK)

    def group(g, carry):
        for b in range(NBUF):
            off = (g * NBUF + b) * CHUNK
            wait_gather(b, off)

            def scale_row(r, c):
                for j in range(D_MODEL // LANES):
                    sl = (r, pl.ds(j * LANES, LANES))
                    rows[b][sl] = rows[b][sl] * SCALE
                return c

            lax.fori_loop(0, CHUNK, scale_row, 0, unroll=2)
            start_scatter(b, off)

            nxt = off + NBUF * CHUNK

            @pl.when(g * NBUF + b + NBUF < NCHUNK)
            def _():
                wait_scatter(b, off)
                start_gather(b, nxt)
        return carry

    lax.fori_loop(0, NCHUNK // NBUF, group, 0)

    for b in range(NBUF):
        off = (NCHUNK - NBUF + b) * CHUNK
        wait_scatter(b, off)


@jax.jit
def _emb(x_flat, lut_pad):
    mesh = plsc.VectorSubcoreMesh(core_axis_name="c", subcore_axis_name="s")
    k = functools.partial(
        pl.kernel,
        mesh=mesh,
        out_type=jax.ShapeDtypeStruct((B_TOTAL, DPAD), jnp.float32),
        compiler_params=pltpu.CompilerParams(use_tc_tiling_on_sc=True),
        scratch_types=[
            pltpu.VMEM((BPW,), jnp.int32),
            pltpu.VMEM((CHUNK, DPAD), jnp.float32),
            pltpu.VMEM((CHUNK, DPAD), jnp.float32),
            pltpu.VMEM((CHUNK, DPAD), jnp.float32),
            pltpu.VMEM((CHUNK, DPAD), jnp.float32),
            pltpu.SemaphoreType.DMA,
            pltpu.SemaphoreType.DMA,
            pltpu.SemaphoreType.DMA,
            pltpu.SemaphoreType.DMA,
            pltpu.SemaphoreType.DMA,
            pltpu.SemaphoreType.DMA,
            pltpu.SemaphoreType.DMA,
            pltpu.SemaphoreType.DMA,
        ],
    )(_emb_body)
    return k(x_flat, lut_pad)


def kernel(x, lut):
    xf = x.reshape(-1).astype(jnp.int32)
    lut_pad = jnp.pad(lut, ((0, 0), (0, DPAD - D_MODEL)))
    out = _emb(xf, lut_pad)
    return out[:, :D_MODEL].reshape(x.shape[0], x.shape[1], D_MODEL)
